# baseline (speedup 1.0000x reference)
import functools

import jax
import jax.numpy as jnp
from jax import lax
from jax.experimental import pallas as pl
from jax.experimental.pallas import tpu as pltpu
from jax.experimental.pallas import tpu_sc as plsc

_NUM_STEPS = 1000
_DIM = 128
_BATCH = 16384

_TC_BATCH = 4096
_SC_BATCH = _BATCH - _TC_BATCH

_info = plsc.get_sparse_core_info()
_NC, _NS = _info.num_cores, _info.num_subcores
_NW = _NC * _NS
_BPW = _SC_BATCH // _NW
_CHUNK = 128
_NCHUNK = _BPW // _CHUNK

_TC_BLOCK = 512
_TC_GRID = _TC_BATCH // _TC_BLOCK
_TPAD = 1024


def _sc_gather_kernel(table_hbm, idx_hbm, out_hbm, idx_v, rows_v, sem):
    wid = lax.axis_index("s") * _NC + lax.axis_index("c")
    base = wid * _BPW
    pltpu.sync_copy(idx_hbm.at[wid], idx_v)
    gathers = []
    for j in range(_NCHUNK):
        gathers.append(
            pltpu.async_copy(
                table_hbm.at[idx_v.at[j]],
                rows_v.at[pl.ds(j * _CHUNK, _CHUNK)],
                sem,
            )
        )
    for g in gathers:
        g.wait()
    pltpu.sync_copy(rows_v, out_hbm.at[pl.ds(base, _BPW)])


def _tc_onehot_kernel(idx_ref, table_ref, out_ref):
    t = table_ref[...]
    hi = t.astype(jnp.bfloat16)
    lo = (t - hi.astype(jnp.float32)).astype(jnp.bfloat16)
    idx = idx_ref[0, 0, :]
    iota = lax.broadcasted_iota(jnp.int32, (_TC_BLOCK, _TPAD), 1)
    onehot = (iota == idx[:, None]).astype(jnp.bfloat16)
    out_ref[...] = jnp.dot(
        onehot, hi, preferred_element_type=jnp.float32
    ) + jnp.dot(onehot, lo, preferred_element_type=jnp.float32)


@jax.jit
def _lookup(input, table):
    idx_sc = input[:_SC_BATCH].reshape(_NW, _NCHUNK, _CHUNK)
    idx_tc = input[_SC_BATCH:].reshape(_TC_GRID, 1, _TC_BLOCK)
    table_pad = jnp.pad(table, ((0, _TPAD - _NUM_STEPS), (0, 0)))

    mesh = plsc.VectorSubcoreMesh(core_axis_name="c", subcore_axis_name="s")
    sc_out = pl.kernel(
        _sc_gather_kernel,
        mesh=mesh,
        out_type=jax.ShapeDtypeStruct((_BATCH, _DIM), jnp.float32),
        scratch_types=[
            pltpu.VMEM((_NCHUNK, _CHUNK), jnp.int32),
            pltpu.VMEM((_BPW, _DIM), jnp.float32),
            pltpu.SemaphoreType.DMA,
        ],
    )(table, idx_sc)

    tc_out = pl.pallas_call(
        _tc_onehot_kernel,
        grid=(_TC_GRID,),
        in_specs=[
            pl.BlockSpec((1, 1, _TC_BLOCK), lambda i: (i, 0, 0)),
            pl.BlockSpec((_TPAD, _DIM), lambda i: (0, 0)),
        ],
        out_specs=pl.BlockSpec((_TC_BLOCK, _DIM), lambda i: (i, 0)),
        out_shape=jax.ShapeDtypeStruct((_TC_BATCH, _DIM), jnp.float32),
    )(idx_tc, table_pad)

    return lax.dynamic_update_slice(sc_out, tc_out, (_SC_BATCH, 0))


def kernel(input, table):
    return _lookup(input, table)


# speedup vs baseline: 1.0496x; 1.0496x over previous
import functools

import jax
import jax.numpy as jnp
from jax import lax
from jax.experimental import pallas as pl
from jax.experimental.pallas import tpu as pltpu
from jax.experimental.pallas import tpu_sc as plsc

_NUM_STEPS = 1000
_DIM = 128
_BATCH = 16384

_info = plsc.get_sparse_core_info()
_NC, _NS = _info.num_cores, _info.num_subcores
_NW = _NC * _NS
_BPW = _BATCH // _NW
_CHUNK = 128
_NCHUNK = _BPW // _CHUNK


def _gather_kernel(table_hbm, idx_hbm, out_hbm, idx_v, rows_v, gsem, *isems):
    wid = lax.axis_index("s") * _NC + lax.axis_index("c")
    base = wid * _BPW
    idx_copies = []
    for j in range(_NCHUNK):
        idx_copies.append(
            pltpu.async_copy(idx_hbm.at[wid, j], idx_v.at[j], isems[j])
        )
    gathers = []
    for j in range(_NCHUNK):
        idx_copies[j].wait()
        gathers.append(
            pltpu.async_copy(
                table_hbm.at[idx_v.at[j]],
                rows_v.at[pl.ds(j * _CHUNK, _CHUNK)],
                gsem,
            )
        )
    for g in gathers:
        g.wait()
    pltpu.sync_copy(rows_v, out_hbm.at[pl.ds(base, _BPW)])


@jax.jit
def _lookup(input, table):
    idx3 = input.reshape(_NW, _NCHUNK, _CHUNK)
    mesh = plsc.VectorSubcoreMesh(core_axis_name="c", subcore_axis_name="s")
    return pl.kernel(
        _gather_kernel,
        mesh=mesh,
        out_type=jax.ShapeDtypeStruct((_BATCH, _DIM), jnp.float32),
        scratch_types=[
            pltpu.VMEM((_NCHUNK, _CHUNK), jnp.int32),
            pltpu.VMEM((_BPW, _DIM), jnp.float32),
            pltpu.SemaphoreType.DMA,
        ] + [pltpu.SemaphoreType.DMA] * _NCHUNK,
    )(table, idx3)


def kernel(input, table):
    return _lookup(input, table)


# speedup vs baseline: 1.1856x; 1.1296x over previous
import functools

import jax
import jax.numpy as jnp
from jax import lax
from jax.experimental import pallas as pl
from jax.experimental.pallas import tpu as pltpu
from jax.experimental.pallas import tpu_sc as plsc

_NUM_STEPS = 1000
_DIM = 128
_BATCH = 16384

_info = plsc.get_sparse_core_info()
_NC, _NS = _info.num_cores, _info.num_subcores
_NW = _NC * _NS
_BPW = _BATCH // _NW
_CHUNK = 128
_NCHUNK = _BPW // _CHUNK

_TROWS = 64


def _gather_kernel(table_hbm, idx_hbm, out_hbm, idx_v, rows_v, tab_s, sem):
    cid = lax.axis_index("c")
    sid = lax.axis_index("s")
    wid = sid * _NC + cid
    base = wid * _BPW
    @pl.when(sid < 15)
    def _stage():
        pltpu.sync_copy(
            table_hbm.at[pl.ds(sid * _TROWS, _TROWS)],
            tab_s.at[pl.ds(sid * _TROWS, _TROWS)],
        )

    @pl.when(sid == 15)
    def _stage_tail():
        pltpu.sync_copy(
            table_hbm.at[pl.ds(15 * _TROWS, _NUM_STEPS - 15 * _TROWS)],
            tab_s.at[pl.ds(15 * _TROWS, _NUM_STEPS - 15 * _TROWS)],
        )

    pltpu.sync_copy(idx_hbm.at[wid], idx_v)
    plsc.subcore_barrier()
    gathers = []
    for j in range(_NCHUNK):
        gathers.append(
            pltpu.async_copy(
                tab_s.at[idx_v.at[j]],
                rows_v.at[pl.ds(j * _CHUNK, _CHUNK)],
                sem,
            )
        )
    for g in gathers:
        g.wait()
    pltpu.sync_copy(rows_v, out_hbm.at[pl.ds(base, _BPW)])


@jax.jit
def _lookup(input, table):
    idx3 = input.reshape(_NW, _NCHUNK, _CHUNK)
    mesh = plsc.VectorSubcoreMesh(core_axis_name="c", subcore_axis_name="s")
    return pl.kernel(
        _gather_kernel,
        mesh=mesh,
        out_type=jax.ShapeDtypeStruct((_BATCH, _DIM), jnp.float32),
        scratch_types=[
            pltpu.VMEM((_NCHUNK, _CHUNK), jnp.int32),
            pltpu.VMEM((_BPW, _DIM), jnp.float32),
            pltpu.VMEM_SHARED((_NUM_STEPS, _DIM), jnp.float32),
            pltpu.SemaphoreType.DMA,
        ],
    )(table, idx3)


def kernel(input, table):
    return _lookup(input, table)


# speedup vs baseline: 1.2124x; 1.0226x over previous
import functools

import jax
import jax.numpy as jnp
from jax import lax
from jax.experimental import pallas as pl
from jax.experimental.pallas import tpu as pltpu
from jax.experimental.pallas import tpu_sc as plsc

_NUM_STEPS = 1000
_DIM = 128
_BATCH = 16384

_info = plsc.get_sparse_core_info()
_NC, _NS = _info.num_cores, _info.num_subcores
_NW = _NC * _NS
_BPW = _BATCH // _NW
_CHUNK = 128
_NCHUNK = _BPW // _CHUNK

_TROWS = 64


def _gather_kernel(table_hbm, idx_hbm, out_hbm, idx_v, rows_v, tab_s, sem, isem):
    cid = lax.axis_index("c")
    sid = lax.axis_index("s")
    wid = sid * _NC + cid
    base = wid * _BPW
    idx_copy = pltpu.async_copy(idx_hbm.at[wid], idx_v, isem)
    @pl.when(sid < 15)
    def _stage():
        pltpu.sync_copy(
            table_hbm.at[pl.ds(sid * _TROWS, _TROWS)],
            tab_s.at[pl.ds(sid * _TROWS, _TROWS)],
        )

    @pl.when(sid == 15)
    def _stage_tail():
        pltpu.sync_copy(
            table_hbm.at[pl.ds(15 * _TROWS, _NUM_STEPS - 15 * _TROWS)],
            tab_s.at[pl.ds(15 * _TROWS, _NUM_STEPS - 15 * _TROWS)],
        )

    idx_copy.wait()
    plsc.subcore_barrier()
    gathers = []
    for j in range(_NCHUNK):
        gathers.append(
            pltpu.async_copy(
                tab_s.at[idx_v.at[j]],
                rows_v.at[pl.ds(j * _CHUNK, _CHUNK)],
                sem,
            )
        )
    for g in gathers:
        g.wait()
    pltpu.sync_copy(rows_v, out_hbm.at[pl.ds(base, _BPW)])


@jax.jit
def _lookup(input, table):
    idx3 = input.reshape(_NW, _NCHUNK, _CHUNK)
    mesh = plsc.VectorSubcoreMesh(core_axis_name="c", subcore_axis_name="s")
    return pl.kernel(
        _gather_kernel,
        mesh=mesh,
        out_type=jax.ShapeDtypeStruct((_BATCH, _DIM), jnp.float32),
        scratch_types=[
            pltpu.VMEM((_NCHUNK, _CHUNK), jnp.int32),
            pltpu.VMEM((_BPW, _DIM), jnp.float32),
            pltpu.VMEM_SHARED((_NUM_STEPS, _DIM), jnp.float32),
            pltpu.SemaphoreType.DMA,
            pltpu.SemaphoreType.DMA,
        ],
    )(table, idx3)


def kernel(input, table):
    return _lookup(input, table)
